# baseline (speedup 1.0000x reference)
import jax
import jax.numpy as jnp
from jax.experimental import pallas as pl
from jax.experimental.pallas import tpu as pltpu

B, N, DIN, H, DOUT = 16, 512, 128, 64, 64
G = 2


def _gcn_fused_kernel(a_ref, x_ref, w1_ref, b1_ref, w2_ref, b2_ref, out_ref):
    w1b = w1_ref[...].astype(jnp.bfloat16)
    w2b = w2_ref[...].astype(jnp.bfloat16)
    b1c = b1_ref[...][:, None]
    b2c = b2_ref[...][:, None]
    for g in range(G):
        A = a_ref[g]
        x = x_ref[g]

        deg = jnp.sum(A, axis=0, keepdims=True)
        dinv = jnp.where(deg > 0, jax.lax.rsqrt(deg), 0.0)
        Ab = A.astype(jnp.bfloat16)

        xwT = jax.lax.dot_general(w1b, x.astype(jnp.bfloat16),
                                  (((0,), (1,)), ((), ())),
                                  preferred_element_type=jnp.float32)

        s1 = (xwT * dinv).astype(jnp.bfloat16)
        t1 = jnp.dot(s1, Ab, preferred_element_type=jnp.float32)
        h1 = jnp.maximum(t1 * dinv + b1c, 0.0)

        hwT = jax.lax.dot_general(w2b, h1.astype(jnp.bfloat16),
                                  (((0,), (0,)), ((), ())),
                                  preferred_element_type=jnp.float32)
        s2 = (hwT * dinv).astype(jnp.bfloat16)
        t2 = jnp.dot(s2, Ab, preferred_element_type=jnp.float32)
        o2 = jnp.maximum(t2 * dinv + b2c, 0.0)

        out_ref[g] = o2.T


def kernel(edge_features, edge_weights, W1, b1, W2, b2):
    return pl.pallas_call(
        _gcn_fused_kernel,
        grid=(B // G,),
        in_specs=[
            pl.BlockSpec((G, N, N), lambda b: (b, 0, 0)),
            pl.BlockSpec((G, N, DIN), lambda b: (b, 0, 0)),
            pl.BlockSpec((DIN, H), lambda b: (0, 0)),
            pl.BlockSpec((H,), lambda b: (0,)),
            pl.BlockSpec((H, DOUT), lambda b: (0, 0)),
            pl.BlockSpec((DOUT,), lambda b: (0,)),
        ],
        out_specs=pl.BlockSpec((G, N, DOUT), lambda b: (b, 0, 0)),
        out_shape=jax.ShapeDtypeStruct((B, N, DOUT), jnp.float32),
        compiler_params=pltpu.CompilerParams(
            dimension_semantics=("parallel",)),
    )(edge_weights, edge_features, W1, b1, W2, b2)


# speedup vs baseline: 1.0830x; 1.0830x over previous
import jax
import jax.numpy as jnp
from jax.experimental import pallas as pl
from jax.experimental.pallas import tpu as pltpu

B, N, DIN, H, DOUT = 16, 512, 128, 64, 64
G = 4


def _gcn_fused_kernel(a_ref, x_ref, w1_ref, b1_ref, w2_ref, b2_ref, out_ref):
    w1b = w1_ref[...].astype(jnp.bfloat16)
    w2b = w2_ref[...].astype(jnp.bfloat16)
    b1c = b1_ref[...][:, None]
    b2c = b2_ref[...][:, None]
    for g in range(G):
        A = a_ref[g]
        x = x_ref[g]

        deg = jnp.sum(A, axis=0, keepdims=True)
        dinv = jnp.where(deg > 0, jax.lax.rsqrt(deg), 0.0)
        Ab = A.astype(jnp.bfloat16)

        xwT = jax.lax.dot_general(w1b, x.astype(jnp.bfloat16),
                                  (((0,), (1,)), ((), ())),
                                  preferred_element_type=jnp.float32)

        s1 = (xwT * dinv).astype(jnp.bfloat16)
        t1 = jnp.dot(s1, Ab, preferred_element_type=jnp.float32)
        h1 = jnp.maximum(t1 * dinv + b1c, 0.0)

        hwT = jax.lax.dot_general(w2b, h1.astype(jnp.bfloat16),
                                  (((0,), (0,)), ((), ())),
                                  preferred_element_type=jnp.float32)
        s2 = (hwT * dinv).astype(jnp.bfloat16)
        t2 = jnp.dot(s2, Ab, preferred_element_type=jnp.float32)
        o2 = jnp.maximum(t2 * dinv + b2c, 0.0)

        out_ref[g] = o2.T


def kernel(edge_features, edge_weights, W1, b1, W2, b2):
    return pl.pallas_call(
        _gcn_fused_kernel,
        grid=(B // G,),
        in_specs=[
            pl.BlockSpec((G, N, N), lambda b: (b, 0, 0)),
            pl.BlockSpec((G, N, DIN), lambda b: (b, 0, 0)),
            pl.BlockSpec((DIN, H), lambda b: (0, 0)),
            pl.BlockSpec((H,), lambda b: (0,)),
            pl.BlockSpec((H, DOUT), lambda b: (0, 0)),
            pl.BlockSpec((DOUT,), lambda b: (0,)),
        ],
        out_specs=pl.BlockSpec((G, N, DOUT), lambda b: (b, 0, 0)),
        out_shape=jax.ShapeDtypeStruct((B, N, DOUT), jnp.float32),
        compiler_params=pltpu.CompilerParams(
            dimension_semantics=("parallel",)),
    )(edge_weights, edge_features, W1, b1, W2, b2)


# speedup vs baseline: 1.2640x; 1.1672x over previous
import jax
import jax.numpy as jnp
from jax.experimental import pallas as pl
from jax.experimental.pallas import tpu as pltpu

B, N, DIN, H, DOUT = 16, 512, 128, 64, 64
G = 4


def _gcn_fused_kernel(a_ref, x_ref, w1_ref, b1_ref, w2_ref, b2_ref, out_ref):
    w1b = w1_ref[...].astype(jnp.bfloat16)
    w2b = w2_ref[...].astype(jnp.bfloat16)
    b1c = b1_ref[...][:, None]
    b2c = b2_ref[...][:, None]
    gs = range(G)
    As = [a_ref[g] for g in gs]
    degs = [jnp.sum(As[g], axis=0, keepdims=True) for g in gs]
    dinvs = [jnp.where(degs[g] > 0, jax.lax.rsqrt(degs[g]), 0.0) for g in gs]
    Abs = [As[g].astype(jnp.bfloat16) for g in gs]
    xwTs = [jax.lax.dot_general(w1b, x_ref[g].astype(jnp.bfloat16),
                                (((0,), (1,)), ((), ())),
                                preferred_element_type=jnp.float32)
            for g in gs]
    s1s = [(xwTs[g] * dinvs[g]).astype(jnp.bfloat16) for g in gs]
    t1s = [jnp.dot(s1s[g], Abs[g], preferred_element_type=jnp.float32)
           for g in gs]
    h1s = [jnp.maximum(t1s[g] * dinvs[g] + b1c, 0.0) for g in gs]
    hwTs = [jax.lax.dot_general(w2b, h1s[g].astype(jnp.bfloat16),
                                (((0,), (0,)), ((), ())),
                                preferred_element_type=jnp.float32)
            for g in gs]
    s2s = [(hwTs[g] * dinvs[g]).astype(jnp.bfloat16) for g in gs]
    t2s = [jnp.dot(s2s[g], Abs[g], preferred_element_type=jnp.float32)
           for g in gs]
    o2s = [jnp.maximum(t2s[g] * dinvs[g] + b2c, 0.0) for g in gs]
    for g in gs:
        out_ref[g] = o2s[g].T


def kernel(edge_features, edge_weights, W1, b1, W2, b2):
    return pl.pallas_call(
        _gcn_fused_kernel,
        grid=(B // G,),
        in_specs=[
            pl.BlockSpec((G, N, N), lambda b: (b, 0, 0)),
            pl.BlockSpec((G, N, DIN), lambda b: (b, 0, 0)),
            pl.BlockSpec((DIN, H), lambda b: (0, 0)),
            pl.BlockSpec((H,), lambda b: (0,)),
            pl.BlockSpec((H, DOUT), lambda b: (0, 0)),
            pl.BlockSpec((DOUT,), lambda b: (0,)),
        ],
        out_specs=pl.BlockSpec((G, N, DOUT), lambda b: (b, 0, 0)),
        out_shape=jax.ShapeDtypeStruct((B, N, DOUT), jnp.float32),
        compiler_params=pltpu.CompilerParams(
            dimension_semantics=("parallel",)),
    )(edge_weights, edge_features, W1, b1, W2, b2)


# speedup vs baseline: 1.3163x; 1.0414x over previous
import jax
import jax.numpy as jnp
from jax.experimental import pallas as pl
from jax.experimental.pallas import tpu as pltpu

B, N, DIN, H, DOUT = 16, 512, 128, 64, 64
G = 8


def _gcn_fused_kernel(a_ref, x_ref, w1_ref, b1_ref, w2_ref, b2_ref, out_ref):
    w1b = w1_ref[...].astype(jnp.bfloat16)
    w2b = w2_ref[...].astype(jnp.bfloat16)
    b1c = b1_ref[...][:, None]
    b2c = b2_ref[...][:, None]
    gs = range(G)
    As = [a_ref[g] for g in gs]
    degs = [jnp.sum(As[g], axis=0, keepdims=True) for g in gs]
    dinvs = [jnp.where(degs[g] > 0, jax.lax.rsqrt(degs[g]), 0.0) for g in gs]
    Abs = [As[g].astype(jnp.bfloat16) for g in gs]
    xwTs = [jax.lax.dot_general(w1b, x_ref[g].astype(jnp.bfloat16),
                                (((0,), (1,)), ((), ())),
                                preferred_element_type=jnp.float32)
            for g in gs]
    s1s = [(xwTs[g] * dinvs[g]).astype(jnp.bfloat16) for g in gs]
    t1s = [jnp.dot(s1s[g], Abs[g], preferred_element_type=jnp.float32)
           for g in gs]
    h1s = [jnp.maximum(t1s[g] * dinvs[g] + b1c, 0.0) for g in gs]
    hwTs = [jax.lax.dot_general(w2b, h1s[g].astype(jnp.bfloat16),
                                (((0,), (0,)), ((), ())),
                                preferred_element_type=jnp.float32)
            for g in gs]
    s2s = [(hwTs[g] * dinvs[g]).astype(jnp.bfloat16) for g in gs]
    t2s = [jnp.dot(s2s[g], Abs[g], preferred_element_type=jnp.float32)
           for g in gs]
    o2s = [jnp.maximum(t2s[g] * dinvs[g] + b2c, 0.0) for g in gs]
    for g in gs:
        out_ref[g] = o2s[g].T


def kernel(edge_features, edge_weights, W1, b1, W2, b2):
    return pl.pallas_call(
        _gcn_fused_kernel,
        grid=(B // G,),
        in_specs=[
            pl.BlockSpec((G, N, N), lambda b: (b, 0, 0)),
            pl.BlockSpec((G, N, DIN), lambda b: (b, 0, 0)),
            pl.BlockSpec((DIN, H), lambda b: (0, 0)),
            pl.BlockSpec((H,), lambda b: (0,)),
            pl.BlockSpec((H, DOUT), lambda b: (0, 0)),
            pl.BlockSpec((DOUT,), lambda b: (0,)),
        ],
        out_specs=pl.BlockSpec((G, N, DOUT), lambda b: (b, 0, 0)),
        out_shape=jax.ShapeDtypeStruct((B, N, DOUT), jnp.float32),
        compiler_params=pltpu.CompilerParams(
            dimension_semantics=("parallel",)),
    )(edge_weights, edge_features, W1, b1, W2, b2)
